# baseline (speedup 1.0000x reference)
import functools

import jax
import jax.numpy as jnp
from jax import lax
from jax.experimental import pallas as pl
from jax.experimental.pallas import tpu as pltpu
from jax.experimental.pallas import tpu_sc as plsc

_K = 8
_EPS_D = 1e-12
_EPS_W = 1e-08
_DISP_SCALE = 0.3


def _topk_body(tmpl_ref, surf_ref, idx_ref, w_ref):
    b = pl.program_id(0)
    S = surf_ref.shape[2]
    t = tmpl_ref[0]
    s3 = surf_ref[0]
    sx, sy, sz = s3[0:1, :], s3[1:2, :], s3[2:3, :]
    tx, ty, tz = t[:, 0:1], t[:, 1:2], t[:, 2:3]
    tsq = tx * tx + ty * ty + tz * tz
    ssq = sx * sx + sy * sy + sz * sz
    cross = jax.lax.dot(t.astype(jnp.bfloat16), s3.astype(jnp.bfloat16),
                        preferred_element_type=jnp.float32)
    d2 = (tsq + ssq) - 2.0 * cross

    iota = jax.lax.broadcasted_iota(jnp.int32, d2.shape, 1).astype(jnp.float32)
    key = d2
    ms, cs = [], []
    for k in range(_K):
        m = jnp.min(key, axis=1, keepdims=True)
        eq = key == m
        c = jnp.min(jnp.where(eq, iota, float(S)), axis=1, keepdims=True)
        ms.append(m)
        cs.append(c)
        if k < _K - 1:
            key = jnp.where(eq, jnp.inf, key)

    mvals = jnp.concatenate(ms, axis=1)
    cols = jnp.concatenate(cs, axis=1).astype(jnp.int32)
    dist = jnp.sqrt(jnp.maximum(mvals, _EPS_D))
    w = 1.0 / (dist + _EPS_W)
    w = w * (1.0 / jnp.sum(w, axis=1, keepdims=True))
    idx_ref[0] = cols + b * S
    w_ref[0] = w


def _topk(template, surf_t, RT):
    B, T, _ = template.shape
    S = surf_t.shape[2]
    grid = (B, T // RT)
    idx, w = pl.pallas_call(
        _topk_body,
        grid=grid,
        in_specs=[
            pl.BlockSpec((1, RT, 3), lambda b, i: (b, i, 0)),
            pl.BlockSpec((1, 3, S), lambda b, i: (b, 0, 0)),
        ],
        out_specs=[
            pl.BlockSpec((1, RT, _K), lambda b, i: (b, i, 0)),
            pl.BlockSpec((1, RT, _K), lambda b, i: (b, i, 0)),
        ],
        out_shape=[
            jax.ShapeDtypeStruct((B, T, _K), jnp.int32),
            jax.ShapeDtypeStruct((B, T, _K), jnp.float32),
        ],
    )(template, surf_t)
    return idx, w


def _sc_gather(pf_flat, idx_flat, w_flat, N, LD):
    info = plsc.get_sparse_core_info()
    NC, NS, L = info.num_cores, info.num_subcores, info.num_lanes
    NW = NC * NS
    assert N % NW == 0
    npw = N // NW
    P = 8
    assert npw % P == 0
    nchunks = npw // P
    mesh = plsc.VectorSubcoreMesh(core_axis_name="c", subcore_axis_name="s")

    @functools.partial(
        pl.kernel,
        mesh=mesh,
        out_type=jax.ShapeDtypeStruct((N, LD), jnp.float32),
        scratch_types=[
            pltpu.VMEM((P * _K,), jnp.int32),
            pltpu.VMEM((P * _K,), jnp.float32),
            pltpu.VMEM((P * _K, LD), jnp.float32),
            pltpu.VMEM((P, LD), jnp.float32),
            pltpu.SemaphoreType.DMA,
        ],
    )
    def k(pf_hbm, idx_hbm, w_hbm, out_hbm, idx_v, w_v, rows_v, out_v, sem):
        wid = lax.axis_index("s") * NC + lax.axis_index("c")
        base = wid * npw

        def chunk(ci, _):
            off = (base + ci * P) * _K
            pltpu.sync_copy(idx_hbm.at[pl.ds(off, P * _K)], idx_v)
            pltpu.sync_copy(w_hbm.at[pl.ds(off, P * _K)], w_v)
            pltpu.async_copy(pf_hbm.at[idx_v], rows_v, sem).wait()

            def pair(q, _):
                wv = w_v[pl.ds(q * (2 * _K), 2 * _K)]
                for half in range(2):
                    p = q * 2 + half
                    for j in range(LD // L):
                        sl = pl.ds(j * L, L)
                        acc = wv[half * _K] * rows_v[p * _K, sl]
                        for kk in range(1, _K):
                            acc = acc + wv[half * _K + kk] * rows_v[p * _K + kk, sl]
                        out_v[p, sl] = acc
                return 0

            lax.fori_loop(0, P // 2, pair, 0)
            pltpu.sync_copy(out_v, out_hbm.at[pl.ds(base + ci * P, P)])
            return 0

        lax.fori_loop(0, nchunks, chunk, 0)

    return k(pf_flat, idx_flat, w_flat)


def _mlp_body(tmpl_ref, loc_ref, gvec_ref,
              w1t_ref, w1l_ref, w1g_ref, b1_ref,
              w2_ref, b2_ref,
              wst_ref, wsl_ref, wsg_ref, bs_ref,
              wot_ref, bo_ref,
              wm1t_ref, wm1l_ref, wm1g_ref, bm1_ref,
              wm2_ref, bm2_ref, wm3_ref, bm3_ref,
              disp_ref, mat_ref):
    t = tmpl_ref[0]
    loc = loc_ref[0]

    def dotf(a, b):
        return jax.lax.dot(a.astype(jnp.bfloat16), b.astype(jnp.bfloat16),
                           preferred_element_type=jnp.float32)

    g = gvec_ref[0]
    gb1 = dotf(g, w1g_ref[...]) + b1_ref[...]
    gbs = dotf(g, wsg_ref[...]) + bs_ref[...]
    gbm = dotf(g, wm1g_ref[...]) + bm1_ref[...]

    h1 = jax.nn.relu(dotf(t, w1t_ref[...]) + dotf(loc, w1l_ref[...]) + gb1)
    h2 = (jax.nn.relu(dotf(h1, w2_ref[...]) + b2_ref[...])
          + dotf(t, wst_ref[...]) + dotf(loc, wsl_ref[...]) + gbs)
    disp_ref[0] = (dotf(h2, wot_ref[...]) + bo_ref[...]) * _DISP_SCALE

    m1 = jax.nn.relu(dotf(t, wm1t_ref[...]) + dotf(loc, wm1l_ref[...]) + gbm)
    m2 = jax.nn.relu(dotf(m1, wm2_ref[...]) + bm2_ref[...])
    z = dotf(m2, wm3_ref[...]) + bm3_ref[...]
    mat_ref[0] = 1.0 / (1.0 + jnp.exp(-z))


def _mlp(template, local_feat, global_feat, params, RT):
    B, T, _ = template.shape
    LD = local_feat.shape[2]
    G = global_feat.shape[1]
    (W1, b1, W2, b2, Wskip, bskip, Wout, bout,
     Wm1, bm1, Wm2, bm2, Wm3, bm3) = params
    H = W1.shape[0]
    HM = Wm1.shape[0]
    HM2 = Wm2.shape[0]

    w1t, w1l, w1g = W1[:, :3].T, W1[:, 3:3 + LD].T, W1[:, 3 + LD:].T
    wst, wsl, wsg = Wskip[:, :3].T, Wskip[:, 3:3 + LD].T, Wskip[:, 3 + LD:].T
    wm1t, wm1l, wm1g = Wm1[:, :3].T, Wm1[:, 3:3 + LD].T, Wm1[:, 3 + LD:].T
    w2, wot, wm2, wm3 = W2.T, Wout.T, Wm2.T, Wm3.T

    def row2(x):
        return x.reshape(1, -1)

    def full_spec(shape):
        return pl.BlockSpec(shape, lambda b, i: (0,) * len(shape))

    grid = (B, T // RT)
    in_specs = [
        pl.BlockSpec((1, RT, 3), lambda b, i: (b, i, 0)),
        pl.BlockSpec((1, RT, LD), lambda b, i: (b, i, 0)),
        pl.BlockSpec((1, 1, G), lambda b, i: (b, 0, 0)),
        full_spec((3, H)), full_spec((LD, H)), full_spec((G, H)), full_spec((1, H)),
        full_spec((H, H)), full_spec((1, H)),
        full_spec((3, H)), full_spec((LD, H)), full_spec((G, H)), full_spec((1, H)),
        full_spec((H, 3)), full_spec((1, 3)),
        full_spec((3, HM)), full_spec((LD, HM)), full_spec((G, HM)), full_spec((1, HM)),
        full_spec((HM, HM2)), full_spec((1, HM2)),
        full_spec((HM2, 1)), full_spec((1, 1)),
    ]
    out_specs = [
        pl.BlockSpec((1, RT, 3), lambda b, i: (b, i, 0)),
        pl.BlockSpec((1, RT, 1), lambda b, i: (b, i, 0)),
    ]
    out_shape = [
        jax.ShapeDtypeStruct((B, T, 3), jnp.float32),
        jax.ShapeDtypeStruct((B, T, 1), jnp.float32),
    ]
    disp, mat = pl.pallas_call(
        _mlp_body, grid=grid, in_specs=in_specs, out_specs=out_specs,
        out_shape=out_shape,
    )(template, local_feat, global_feat[:, None, :],
      w1t, w1l, w1g, row2(b1),
      w2, row2(b2),
      wst, wsl, wsg, row2(bskip),
      wot, row2(bout),
      wm1t, wm1l, wm1g, row2(bm1),
      wm2, row2(bm2), wm3, row2(bm3))
    return disp, mat[..., 0]


def kernel(template, surf_xyz, global_feat, point_feat, W1, b1, W2, b2,
           Wskip, bskip, Wout, bout, Wm1, bm1, Wm2, bm2, Wm3, bm3):
    B, T, _ = template.shape
    S = surf_xyz.shape[1]
    LD = point_feat.shape[2]
    params = (W1, b1, W2, b2, Wskip, bskip, Wout, bout,
              Wm1, bm1, Wm2, bm2, Wm3, bm3)

    surf_t = surf_xyz.transpose(0, 2, 1)
    idx, w = _topk(template, surf_t, RT=256)

    local = _sc_gather(point_feat.reshape(B * S, LD),
                       idx.reshape(-1), w.reshape(-1), B * T, LD)
    local = local.reshape(B, T, LD)

    return _mlp(template, local, global_feat, params, RT=512)


# speedup vs baseline: 1.1560x; 1.1560x over previous
import functools

import jax
import jax.numpy as jnp
from jax import lax
from jax.experimental import pallas as pl
from jax.experimental.pallas import tpu as pltpu
from jax.experimental.pallas import tpu_sc as plsc

_K = 8
_EPS_D = 1e-12
_EPS_W = 1e-08
_DISP_SCALE = 0.3


def _topk_body(tmpl_ref, surf_ref, idx_ref, w_ref):
    b = pl.program_id(0)
    S = surf_ref.shape[2]
    t = tmpl_ref[0]
    s3 = surf_ref[0]
    sx, sy, sz = s3[0:1, :], s3[1:2, :], s3[2:3, :]
    tx, ty, tz = t[:, 0:1], t[:, 1:2], t[:, 2:3]
    tsq = tx * tx + ty * ty + tz * tz
    ssq = sx * sx + sy * sy + sz * sz
    cross = jax.lax.dot(t.astype(jnp.bfloat16), s3.astype(jnp.bfloat16),
                        preferred_element_type=jnp.float32)
    d2 = (tsq + ssq) - 2.0 * cross

    iota = jax.lax.broadcasted_iota(jnp.int32, d2.shape, 1).astype(jnp.float32)
    key = d2
    ms, cs = [], []
    for k in range(_K):
        m = jnp.min(key, axis=1, keepdims=True)
        eq = key == m
        c = jnp.min(jnp.where(eq, iota, float(S)), axis=1, keepdims=True)
        ms.append(m)
        cs.append(c)
        if k < _K - 1:
            key = jnp.where(eq, jnp.inf, key)

    mvals = jnp.concatenate(ms, axis=1)
    cols = jnp.concatenate(cs, axis=1).astype(jnp.int32)
    dist = jnp.sqrt(jnp.maximum(mvals, _EPS_D))
    w = 1.0 / (dist + _EPS_W)
    w = w * (1.0 / jnp.sum(w, axis=1, keepdims=True))
    idx_ref[0] = cols + b * S
    w_ref[0] = w


def _topk(template, surf_t, RT):
    B, T, _ = template.shape
    S = surf_t.shape[2]
    grid = (B, T // RT)
    idx, w = pl.pallas_call(
        _topk_body,
        grid=grid,
        in_specs=[
            pl.BlockSpec((1, RT, 3), lambda b, i: (b, i, 0)),
            pl.BlockSpec((1, 3, S), lambda b, i: (b, 0, 0)),
        ],
        out_specs=[
            pl.BlockSpec((1, RT, _K), lambda b, i: (b, i, 0)),
            pl.BlockSpec((1, RT, _K), lambda b, i: (b, i, 0)),
        ],
        out_shape=[
            jax.ShapeDtypeStruct((B, T, _K), jnp.int32),
            jax.ShapeDtypeStruct((B, T, _K), jnp.float32),
        ],
    )(template, surf_t)
    return idx, w


def _sc_gather(pf_flat, idx_flat, w_flat, N, LD):
    info = plsc.get_sparse_core_info()
    NC, NS, L = info.num_cores, info.num_subcores, info.num_lanes
    NW = NC * NS
    assert N % NW == 0
    npw = N // NW
    P = 16
    assert npw % (2 * P) == 0
    nchunks = npw // P
    mesh = plsc.VectorSubcoreMesh(core_axis_name="c", subcore_axis_name="s")

    @functools.partial(
        pl.kernel,
        mesh=mesh,
        out_type=jax.ShapeDtypeStruct((N, LD), jnp.float32),
        scratch_types=[
            pltpu.VMEM((2, P * _K), jnp.int32),
            pltpu.VMEM((2, P * _K), jnp.float32),
            pltpu.VMEM((2, P * _K, LD), jnp.float32),
            pltpu.VMEM((P, LD), jnp.float32),
            pltpu.SemaphoreType.DMA,
            pltpu.SemaphoreType.DMA,
        ],
    )
    def k(pf_hbm, idx_hbm, w_hbm, out_hbm, idx_v, w_v, rows_v, out_v,
          sg0, sg1):
        wid = lax.axis_index("s") * NC + lax.axis_index("c")
        base = wid * npw
        sgs = (sg0, sg1)

        def fire(g, buf):
            off = (base + g * P) * _K
            pltpu.sync_copy(idx_hbm.at[pl.ds(off, P * _K)], idx_v.at[buf])
            pltpu.sync_copy(w_hbm.at[pl.ds(off, P * _K)], w_v.at[buf])
            pltpu.async_copy(pf_hbm.at[idx_v.at[buf]], rows_v.at[buf],
                             sgs[buf])

        fire(0, 0)
        fire(1, 1)

        def outer(half_i, _):
            for buf in range(2):
                g = half_i * 2 + buf
                pltpu.make_async_copy(pf_hbm.at[idx_v.at[buf]],
                                      rows_v.at[buf], sgs[buf]).wait()

                def pair(q, _):
                    wv = w_v[buf, pl.ds(q * (2 * _K), 2 * _K)]
                    for half in range(2):
                        p = q * 2 + half
                        for j in range(LD // L):
                            sl = pl.ds(j * L, L)
                            acc = wv[half * _K] * rows_v[buf, p * _K, sl]
                            for kk in range(1, _K):
                                acc = acc + (wv[half * _K + kk]
                                             * rows_v[buf, p * _K + kk, sl])
                            out_v[p, sl] = acc
                    return 0

                lax.fori_loop(0, P // 2, pair, 0)
                pltpu.sync_copy(out_v, out_hbm.at[pl.ds(base + g * P, P)])

                @pl.when(g + 2 < nchunks)
                def _():
                    fire(g + 2, buf)
            return 0

        lax.fori_loop(0, nchunks // 2, outer, 0)

    return k(pf_flat, idx_flat, w_flat)


def _mlp_body(tmpl_ref, loc_ref, gvec_ref,
              w1t_ref, w1l_ref, w1g_ref, b1_ref,
              w2_ref, b2_ref,
              wst_ref, wsl_ref, wsg_ref, bs_ref,
              wot_ref, bo_ref,
              wm1t_ref, wm1l_ref, wm1g_ref, bm1_ref,
              wm2_ref, bm2_ref, wm3_ref, bm3_ref,
              disp_ref, mat_ref):
    t = tmpl_ref[0]
    loc = loc_ref[0]

    def dotf(a, b):
        return jax.lax.dot(a.astype(jnp.bfloat16), b.astype(jnp.bfloat16),
                           preferred_element_type=jnp.float32)

    g = gvec_ref[0]
    gb1 = dotf(g, w1g_ref[...]) + b1_ref[...]
    gbs = dotf(g, wsg_ref[...]) + bs_ref[...]
    gbm = dotf(g, wm1g_ref[...]) + bm1_ref[...]

    h1 = jax.nn.relu(dotf(t, w1t_ref[...]) + dotf(loc, w1l_ref[...]) + gb1)
    h2 = (jax.nn.relu(dotf(h1, w2_ref[...]) + b2_ref[...])
          + dotf(t, wst_ref[...]) + dotf(loc, wsl_ref[...]) + gbs)
    disp_ref[0] = (dotf(h2, wot_ref[...]) + bo_ref[...]) * _DISP_SCALE

    m1 = jax.nn.relu(dotf(t, wm1t_ref[...]) + dotf(loc, wm1l_ref[...]) + gbm)
    m2 = jax.nn.relu(dotf(m1, wm2_ref[...]) + bm2_ref[...])
    z = dotf(m2, wm3_ref[...]) + bm3_ref[...]
    mat_ref[0] = 1.0 / (1.0 + jnp.exp(-z))


def _mlp(template, local_feat, global_feat, params, RT):
    B, T, _ = template.shape
    LD = local_feat.shape[2]
    G = global_feat.shape[1]
    (W1, b1, W2, b2, Wskip, bskip, Wout, bout,
     Wm1, bm1, Wm2, bm2, Wm3, bm3) = params
    H = W1.shape[0]
    HM = Wm1.shape[0]
    HM2 = Wm2.shape[0]

    w1t, w1l, w1g = W1[:, :3].T, W1[:, 3:3 + LD].T, W1[:, 3 + LD:].T
    wst, wsl, wsg = Wskip[:, :3].T, Wskip[:, 3:3 + LD].T, Wskip[:, 3 + LD:].T
    wm1t, wm1l, wm1g = Wm1[:, :3].T, Wm1[:, 3:3 + LD].T, Wm1[:, 3 + LD:].T
    w2, wot, wm2, wm3 = W2.T, Wout.T, Wm2.T, Wm3.T

    def row2(x):
        return x.reshape(1, -1)

    def full_spec(shape):
        return pl.BlockSpec(shape, lambda b, i: (0,) * len(shape))

    grid = (B, T // RT)
    in_specs = [
        pl.BlockSpec((1, RT, 3), lambda b, i: (b, i, 0)),
        pl.BlockSpec((1, RT, LD), lambda b, i: (b, i, 0)),
        pl.BlockSpec((1, 1, G), lambda b, i: (b, 0, 0)),
        full_spec((3, H)), full_spec((LD, H)), full_spec((G, H)), full_spec((1, H)),
        full_spec((H, H)), full_spec((1, H)),
        full_spec((3, H)), full_spec((LD, H)), full_spec((G, H)), full_spec((1, H)),
        full_spec((H, 3)), full_spec((1, 3)),
        full_spec((3, HM)), full_spec((LD, HM)), full_spec((G, HM)), full_spec((1, HM)),
        full_spec((HM, HM2)), full_spec((1, HM2)),
        full_spec((HM2, 1)), full_spec((1, 1)),
    ]
    out_specs = [
        pl.BlockSpec((1, RT, 3), lambda b, i: (b, i, 0)),
        pl.BlockSpec((1, RT, 1), lambda b, i: (b, i, 0)),
    ]
    out_shape = [
        jax.ShapeDtypeStruct((B, T, 3), jnp.float32),
        jax.ShapeDtypeStruct((B, T, 1), jnp.float32),
    ]
    disp, mat = pl.pallas_call(
        _mlp_body, grid=grid, in_specs=in_specs, out_specs=out_specs,
        out_shape=out_shape,
    )(template, local_feat, global_feat[:, None, :],
      w1t, w1l, w1g, row2(b1),
      w2, row2(b2),
      wst, wsl, wsg, row2(bskip),
      wot, row2(bout),
      wm1t, wm1l, wm1g, row2(bm1),
      wm2, row2(bm2), wm3, row2(bm3))
    return disp, mat[..., 0]


def kernel(template, surf_xyz, global_feat, point_feat, W1, b1, W2, b2,
           Wskip, bskip, Wout, bout, Wm1, bm1, Wm2, bm2, Wm3, bm3):
    B, T, _ = template.shape
    S = surf_xyz.shape[1]
    LD = point_feat.shape[2]
    params = (W1, b1, W2, b2, Wskip, bskip, Wout, bout,
              Wm1, bm1, Wm2, bm2, Wm3, bm3)

    surf_t = surf_xyz.transpose(0, 2, 1)
    idx, w = _topk(template, surf_t, RT=256)

    local = _sc_gather(point_feat.reshape(B * S, LD),
                       idx.reshape(-1), w.reshape(-1), B * T, LD)
    local = local.reshape(B, T, LD)

    return _mlp(template, local, global_feat, params, RT=512)


# speedup vs baseline: 1.4555x; 1.2590x over previous
import functools

import jax
import jax.numpy as jnp
from jax import lax
from jax.experimental import pallas as pl
from jax.experimental.pallas import tpu as pltpu
from jax.experimental.pallas import tpu_sc as plsc

_K = 8
_EPS_D = 1e-12
_EPS_W = 1e-08
_DISP_SCALE = 0.3


def _topk_body(tmpl_ref, surf_ref, idx_ref, w_ref):
    S = surf_ref.shape[2]
    t = tmpl_ref[0]
    s3 = surf_ref[0]
    sx, sy, sz = s3[0:1, :], s3[1:2, :], s3[2:3, :]
    tx, ty, tz = t[:, 0:1], t[:, 1:2], t[:, 2:3]
    tsq = tx * tx + ty * ty + tz * tz
    ssq = sx * sx + sy * sy + sz * sz
    cross = jax.lax.dot(t.astype(jnp.bfloat16), s3.astype(jnp.bfloat16),
                        preferred_element_type=jnp.float32)
    d2 = (tsq + ssq) - 2.0 * cross

    iota = jax.lax.broadcasted_iota(jnp.int32, d2.shape, 1).astype(jnp.float32)
    key = d2
    ms, cs = [], []
    for k in range(_K):
        m = jnp.min(key, axis=1, keepdims=True)
        eq = key == m
        c = jnp.min(jnp.where(eq, iota, float(S)), axis=1, keepdims=True)
        ms.append(m)
        cs.append(c)
        if k < _K - 1:
            key = jnp.where(eq, jnp.inf, key)

    mvals = jnp.concatenate(ms, axis=1)
    cols = jnp.concatenate(cs, axis=1).astype(jnp.int32)
    dist = jnp.sqrt(jnp.maximum(mvals, _EPS_D))
    w = 1.0 / (dist + _EPS_W)
    w = w * (1.0 / jnp.sum(w, axis=1, keepdims=True))
    idx_ref[0] = cols
    w_ref[0] = w


def _topk(template, surf_t, RT):
    T = template.shape[1]
    S = surf_t.shape[2]
    grid = (T // RT,)
    idx, w = pl.pallas_call(
        _topk_body,
        grid=grid,
        in_specs=[
            pl.BlockSpec((1, RT, 3), lambda i: (0, i, 0)),
            pl.BlockSpec((1, 3, S), lambda i: (0, 0, 0)),
        ],
        out_specs=[
            pl.BlockSpec((1, RT, _K), lambda i: (0, i, 0)),
            pl.BlockSpec((1, RT, _K), lambda i: (0, i, 0)),
        ],
        out_shape=[
            jax.ShapeDtypeStruct((1, T, _K), jnp.int32),
            jax.ShapeDtypeStruct((1, T, _K), jnp.float32),
        ],
    )(template, surf_t)
    return idx, w


def _sc_gather(pf_flat, idx_flat, w_flat, N, LD):
    info = plsc.get_sparse_core_info()
    NC, NS, L = info.num_cores, info.num_subcores, info.num_lanes
    NW = NC * NS
    assert N % NW == 0
    npw = N // NW
    P = 16
    assert npw % (2 * P) == 0
    nchunks = npw // P
    mesh = plsc.VectorSubcoreMesh(core_axis_name="c", subcore_axis_name="s")

    @functools.partial(
        pl.kernel,
        mesh=mesh,
        out_type=jax.ShapeDtypeStruct((N, LD), jnp.float32),
        scratch_types=[
            pltpu.VMEM((2, P * _K), jnp.int32),
            pltpu.VMEM((2, P * _K), jnp.float32),
            pltpu.VMEM((2, P * _K, LD), jnp.float32),
            pltpu.VMEM((P, LD), jnp.float32),
            pltpu.SemaphoreType.DMA,
            pltpu.SemaphoreType.DMA,
        ],
    )
    def k(pf_hbm, idx_hbm, w_hbm, out_hbm, idx_v, w_v, rows_v, out_v,
          sg0, sg1):
        wid = lax.axis_index("s") * NC + lax.axis_index("c")
        base = wid * npw
        sgs = (sg0, sg1)

        def fire(g, buf):
            off = (base + g * P) * _K
            pltpu.sync_copy(idx_hbm.at[pl.ds(off, P * _K)], idx_v.at[buf])
            pltpu.sync_copy(w_hbm.at[pl.ds(off, P * _K)], w_v.at[buf])
            pltpu.async_copy(pf_hbm.at[idx_v.at[buf]], rows_v.at[buf],
                             sgs[buf])

        fire(0, 0)
        fire(1, 1)

        def outer(half_i, _):
            for buf in range(2):
                g = half_i * 2 + buf
                pltpu.make_async_copy(pf_hbm.at[idx_v.at[buf]],
                                      rows_v.at[buf], sgs[buf]).wait()

                def pair(q, _):
                    wv = w_v[buf, pl.ds(q * (2 * _K), 2 * _K)]
                    for half in range(2):
                        p = q * 2 + half
                        for j in range(LD // L):
                            sl = pl.ds(j * L, L)
                            acc = wv[half * _K] * rows_v[buf, p * _K, sl]
                            for kk in range(1, _K):
                                acc = acc + (wv[half * _K + kk]
                                             * rows_v[buf, p * _K + kk, sl])
                            out_v[p, sl] = acc
                    return 0

                lax.fori_loop(0, P // 2, pair, 0)
                pltpu.sync_copy(out_v, out_hbm.at[pl.ds(base + g * P, P)])

                @pl.when(g + 2 < nchunks)
                def _():
                    fire(g + 2, buf)
            return 0

        lax.fori_loop(0, nchunks // 2, outer, 0)

    return k(pf_flat, idx_flat, w_flat)


def _mlp_body(tmpl_ref, loc_ref, gvec_ref,
              w1t_ref, w1l_ref, w1g_ref, b1_ref,
              w2_ref, b2_ref,
              wst_ref, wsl_ref, wsg_ref, bs_ref,
              wot_ref, bo_ref,
              wm1t_ref, wm1l_ref, wm1g_ref, bm1_ref,
              wm2_ref, bm2_ref, wm3_ref, bm3_ref,
              disp_ref, mat_ref):
    t = tmpl_ref[0]
    loc = loc_ref[0]

    def dotf(a, b):
        return jax.lax.dot(a.astype(jnp.bfloat16), b.astype(jnp.bfloat16),
                           preferred_element_type=jnp.float32)

    g = gvec_ref[0]
    gb1 = dotf(g, w1g_ref[...]) + b1_ref[...]
    gbs = dotf(g, wsg_ref[...]) + bs_ref[...]
    gbm = dotf(g, wm1g_ref[...]) + bm1_ref[...]

    h1 = jax.nn.relu(dotf(t, w1t_ref[...]) + dotf(loc, w1l_ref[...]) + gb1)
    h2 = (jax.nn.relu(dotf(h1, w2_ref[...]) + b2_ref[...])
          + dotf(t, wst_ref[...]) + dotf(loc, wsl_ref[...]) + gbs)
    disp_ref[0] = (dotf(h2, wot_ref[...]) + bo_ref[...]) * _DISP_SCALE

    m1 = jax.nn.relu(dotf(t, wm1t_ref[...]) + dotf(loc, wm1l_ref[...]) + gbm)
    m2 = jax.nn.relu(dotf(m1, wm2_ref[...]) + bm2_ref[...])
    z = dotf(m2, wm3_ref[...]) + bm3_ref[...]
    mat_ref[0] = 1.0 / (1.0 + jnp.exp(-z))


def _mlp(template, local_feat, global_feat, params, RT):
    B, T, _ = template.shape
    LD = local_feat.shape[2]
    G = global_feat.shape[1]
    (W1, b1, W2, b2, Wskip, bskip, Wout, bout,
     Wm1, bm1, Wm2, bm2, Wm3, bm3) = params
    H = W1.shape[0]
    HM = Wm1.shape[0]
    HM2 = Wm2.shape[0]

    w1t, w1l, w1g = W1[:, :3].T, W1[:, 3:3 + LD].T, W1[:, 3 + LD:].T
    wst, wsl, wsg = Wskip[:, :3].T, Wskip[:, 3:3 + LD].T, Wskip[:, 3 + LD:].T
    wm1t, wm1l, wm1g = Wm1[:, :3].T, Wm1[:, 3:3 + LD].T, Wm1[:, 3 + LD:].T
    w2, wot, wm2, wm3 = W2.T, Wout.T, Wm2.T, Wm3.T

    def row2(x):
        return x.reshape(1, -1)

    def full_spec(shape):
        return pl.BlockSpec(shape, lambda i: (0,) * len(shape))

    grid = (T // RT,)
    in_specs = [
        pl.BlockSpec((1, RT, 3), lambda i: (0, i, 0)),
        pl.BlockSpec((1, RT, LD), lambda i: (0, i, 0)),
        pl.BlockSpec((1, 1, G), lambda i: (0, 0, 0)),
        full_spec((3, H)), full_spec((LD, H)), full_spec((G, H)), full_spec((1, H)),
        full_spec((H, H)), full_spec((1, H)),
        full_spec((3, H)), full_spec((LD, H)), full_spec((G, H)), full_spec((1, H)),
        full_spec((H, 3)), full_spec((1, 3)),
        full_spec((3, HM)), full_spec((LD, HM)), full_spec((G, HM)), full_spec((1, HM)),
        full_spec((HM, HM2)), full_spec((1, HM2)),
        full_spec((HM2, 1)), full_spec((1, 1)),
    ]
    out_specs = [
        pl.BlockSpec((1, RT, 3), lambda i: (0, i, 0)),
        pl.BlockSpec((1, RT, 1), lambda i: (0, i, 0)),
    ]
    out_shape = [
        jax.ShapeDtypeStruct((1, T, 3), jnp.float32),
        jax.ShapeDtypeStruct((1, T, 1), jnp.float32),
    ]
    disp, mat = pl.pallas_call(
        _mlp_body, grid=grid, in_specs=in_specs, out_specs=out_specs,
        out_shape=out_shape,
    )(template, local_feat, global_feat[:, None, :],
      w1t, w1l, w1g, row2(b1),
      w2, row2(b2),
      wst, wsl, wsg, row2(bskip),
      wot, row2(bout),
      wm1t, wm1l, wm1g, row2(bm1),
      wm2, row2(bm2), wm3, row2(bm3))
    return disp, mat[..., 0]


def kernel(template, surf_xyz, global_feat, point_feat, W1, b1, W2, b2,
           Wskip, bskip, Wout, bout, Wm1, bm1, Wm2, bm2, Wm3, bm3):
    B, T, _ = template.shape
    S = surf_xyz.shape[1]
    LD = point_feat.shape[2]
    params = (W1, b1, W2, b2, Wskip, bskip, Wout, bout,
              Wm1, bm1, Wm2, bm2, Wm3, bm3)

    surf_t = surf_xyz.transpose(0, 2, 1)

    tops = [_topk(template[b:b + 1], surf_t[b:b + 1], RT=256)
            for b in range(B)]
    locals_ = [_sc_gather(point_feat[b], idx.reshape(-1), w.reshape(-1),
                          T, LD)[None]
               for b, (idx, w) in enumerate(tops)]
    outs = [_mlp(template[b:b + 1], locals_[b], global_feat[b:b + 1],
                 params, RT=512)
            for b in range(B)]
    disp = jnp.concatenate([o[0] for o in outs], axis=0)
    mat = jnp.concatenate([o[1] for o in outs], axis=0)
    return disp, mat


# speedup vs baseline: 1.5270x; 1.0491x over previous
import functools

import jax
import jax.numpy as jnp
from jax import lax
from jax.experimental import pallas as pl
from jax.experimental.pallas import tpu as pltpu
from jax.experimental.pallas import tpu_sc as plsc

_K = 8
_EPS_D = 1e-12
_EPS_W = 1e-08
_DISP_SCALE = 0.3


def _topk_body(tmpl_ref, surf_ref, idx_ref, w_ref):
    S = surf_ref.shape[2]
    t = tmpl_ref[0]
    s3 = surf_ref[0]
    sx, sy, sz = s3[0:1, :], s3[1:2, :], s3[2:3, :]
    tx, ty, tz = t[:, 0:1], t[:, 1:2], t[:, 2:3]
    tsq = tx * tx + ty * ty + tz * tz
    ssq = sx * sx + sy * sy + sz * sz
    cross = jax.lax.dot(t.astype(jnp.bfloat16), s3.astype(jnp.bfloat16),
                        preferred_element_type=jnp.float32)
    d2 = (tsq + ssq) - 2.0 * cross

    S2 = S // 2
    a = d2[:, :S2]
    b2 = d2[:, S2:]
    cmp = a <= b2
    v = jnp.minimum(a, b2)
    u = jnp.maximum(a, b2)
    iota = jax.lax.broadcasted_iota(jnp.int32, v.shape, 1).astype(jnp.float32)
    iv = jnp.where(cmp, iota, iota + float(S2))
    iu = jnp.where(cmp, iota + float(S2), iota)

    ms, cs = [], []
    for k in range(_K):
        m = jnp.min(v, axis=1, keepdims=True)
        eq = v == m
        c = jnp.min(jnp.where(eq, iv, float(S)), axis=1, keepdims=True)
        ms.append(m)
        cs.append(c)
        if k < _K - 1:
            v = jnp.where(eq, u, v)
            iv = jnp.where(eq, iu, iv)
            u = jnp.where(eq, jnp.inf, u)

    mvals = jnp.concatenate(ms, axis=1)
    cols = jnp.concatenate(cs, axis=1).astype(jnp.int32)
    dist = jnp.sqrt(jnp.maximum(mvals, _EPS_D))
    w = 1.0 / (dist + _EPS_W)
    w = w * (1.0 / jnp.sum(w, axis=1, keepdims=True))
    idx_ref[0] = cols
    w_ref[0] = w


def _topk(template, surf_t, RT):
    T = template.shape[1]
    S = surf_t.shape[2]
    grid = (T // RT,)
    idx, w = pl.pallas_call(
        _topk_body,
        grid=grid,
        in_specs=[
            pl.BlockSpec((1, RT, 3), lambda i: (0, i, 0)),
            pl.BlockSpec((1, 3, S), lambda i: (0, 0, 0)),
        ],
        out_specs=[
            pl.BlockSpec((1, RT, _K), lambda i: (0, i, 0)),
            pl.BlockSpec((1, RT, _K), lambda i: (0, i, 0)),
        ],
        out_shape=[
            jax.ShapeDtypeStruct((1, T, _K), jnp.int32),
            jax.ShapeDtypeStruct((1, T, _K), jnp.float32),
        ],
    )(template, surf_t)
    return idx, w


def _sc_gather(pf_flat, idx_flat, w_flat, N, LD):
    info = plsc.get_sparse_core_info()
    NC, NS, L = info.num_cores, info.num_subcores, info.num_lanes
    NW = NC * NS
    assert N % NW == 0
    npw = N // NW
    P = 16
    assert npw % (2 * P) == 0
    nchunks = npw // P
    mesh = plsc.VectorSubcoreMesh(core_axis_name="c", subcore_axis_name="s")

    @functools.partial(
        pl.kernel,
        mesh=mesh,
        out_type=jax.ShapeDtypeStruct((N, LD), jnp.float32),
        scratch_types=[
            pltpu.VMEM((2, P * _K), jnp.int32),
            pltpu.VMEM((2, P * _K), jnp.float32),
            pltpu.VMEM((2, P * _K, LD), jnp.float32),
            pltpu.VMEM((P, LD), jnp.float32),
            pltpu.SemaphoreType.DMA,
            pltpu.SemaphoreType.DMA,
        ],
    )
    def k(pf_hbm, idx_hbm, w_hbm, out_hbm, idx_v, w_v, rows_v, out_v,
          sg0, sg1):
        wid = lax.axis_index("s") * NC + lax.axis_index("c")
        base = wid * npw
        sgs = (sg0, sg1)

        def fire(g, buf):
            off = (base + g * P) * _K
            pltpu.sync_copy(idx_hbm.at[pl.ds(off, P * _K)], idx_v.at[buf])
            pltpu.sync_copy(w_hbm.at[pl.ds(off, P * _K)], w_v.at[buf])
            pltpu.async_copy(pf_hbm.at[idx_v.at[buf]], rows_v.at[buf],
                             sgs[buf])

        fire(0, 0)
        fire(1, 1)

        def outer(half_i, _):
            for buf in range(2):
                g = half_i * 2 + buf
                pltpu.make_async_copy(pf_hbm.at[idx_v.at[buf]],
                                      rows_v.at[buf], sgs[buf]).wait()

                def pair(q, _):
                    wv = w_v[buf, pl.ds(q * (2 * _K), 2 * _K)]
                    for half in range(2):
                        p = q * 2 + half
                        for j in range(LD // L):
                            sl = pl.ds(j * L, L)
                            acc = wv[half * _K] * rows_v[buf, p * _K, sl]
                            for kk in range(1, _K):
                                acc = acc + (wv[half * _K + kk]
                                             * rows_v[buf, p * _K + kk, sl])
                            out_v[p, sl] = acc
                    return 0

                lax.fori_loop(0, P // 2, pair, 0)
                pltpu.sync_copy(out_v, out_hbm.at[pl.ds(base + g * P, P)])

                @pl.when(g + 2 < nchunks)
                def _():
                    fire(g + 2, buf)
            return 0

        lax.fori_loop(0, nchunks // 2, outer, 0)

    return k(pf_flat, idx_flat, w_flat)


def _mlp_body(tmpl_ref, loc_ref, gvec_ref,
              w1t_ref, w1l_ref, w1g_ref, b1_ref,
              w2_ref, b2_ref,
              wst_ref, wsl_ref, wsg_ref, bs_ref,
              wot_ref, bo_ref,
              wm1t_ref, wm1l_ref, wm1g_ref, bm1_ref,
              wm2_ref, bm2_ref, wm3_ref, bm3_ref,
              disp_ref, mat_ref):
    t = tmpl_ref[0]
    loc = loc_ref[0]

    def dotf(a, b):
        return jax.lax.dot(a.astype(jnp.bfloat16), b.astype(jnp.bfloat16),
                           preferred_element_type=jnp.float32)

    g = gvec_ref[0]
    gb1 = dotf(g, w1g_ref[...]) + b1_ref[...]
    gbs = dotf(g, wsg_ref[...]) + bs_ref[...]
    gbm = dotf(g, wm1g_ref[...]) + bm1_ref[...]

    h1 = jax.nn.relu(dotf(t, w1t_ref[...]) + dotf(loc, w1l_ref[...]) + gb1)
    h2 = (jax.nn.relu(dotf(h1, w2_ref[...]) + b2_ref[...])
          + dotf(t, wst_ref[...]) + dotf(loc, wsl_ref[...]) + gbs)
    disp_ref[0] = (dotf(h2, wot_ref[...]) + bo_ref[...]) * _DISP_SCALE

    m1 = jax.nn.relu(dotf(t, wm1t_ref[...]) + dotf(loc, wm1l_ref[...]) + gbm)
    m2 = jax.nn.relu(dotf(m1, wm2_ref[...]) + bm2_ref[...])
    z = dotf(m2, wm3_ref[...]) + bm3_ref[...]
    mat_ref[0] = 1.0 / (1.0 + jnp.exp(-z))


def _mlp(template, local_feat, global_feat, params, RT):
    B, T, _ = template.shape
    LD = local_feat.shape[2]
    G = global_feat.shape[1]
    (W1, b1, W2, b2, Wskip, bskip, Wout, bout,
     Wm1, bm1, Wm2, bm2, Wm3, bm3) = params
    H = W1.shape[0]
    HM = Wm1.shape[0]
    HM2 = Wm2.shape[0]

    w1t, w1l, w1g = W1[:, :3].T, W1[:, 3:3 + LD].T, W1[:, 3 + LD:].T
    wst, wsl, wsg = Wskip[:, :3].T, Wskip[:, 3:3 + LD].T, Wskip[:, 3 + LD:].T
    wm1t, wm1l, wm1g = Wm1[:, :3].T, Wm1[:, 3:3 + LD].T, Wm1[:, 3 + LD:].T
    w2, wot, wm2, wm3 = W2.T, Wout.T, Wm2.T, Wm3.T

    def row2(x):
        return x.reshape(1, -1)

    def full_spec(shape):
        return pl.BlockSpec(shape, lambda i: (0,) * len(shape))

    grid = (T // RT,)
    in_specs = [
        pl.BlockSpec((1, RT, 3), lambda i: (0, i, 0)),
        pl.BlockSpec((1, RT, LD), lambda i: (0, i, 0)),
        pl.BlockSpec((1, 1, G), lambda i: (0, 0, 0)),
        full_spec((3, H)), full_spec((LD, H)), full_spec((G, H)), full_spec((1, H)),
        full_spec((H, H)), full_spec((1, H)),
        full_spec((3, H)), full_spec((LD, H)), full_spec((G, H)), full_spec((1, H)),
        full_spec((H, 3)), full_spec((1, 3)),
        full_spec((3, HM)), full_spec((LD, HM)), full_spec((G, HM)), full_spec((1, HM)),
        full_spec((HM, HM2)), full_spec((1, HM2)),
        full_spec((HM2, 1)), full_spec((1, 1)),
    ]
    out_specs = [
        pl.BlockSpec((1, RT, 3), lambda i: (0, i, 0)),
        pl.BlockSpec((1, RT, 1), lambda i: (0, i, 0)),
    ]
    out_shape = [
        jax.ShapeDtypeStruct((1, T, 3), jnp.float32),
        jax.ShapeDtypeStruct((1, T, 1), jnp.float32),
    ]
    disp, mat = pl.pallas_call(
        _mlp_body, grid=grid, in_specs=in_specs, out_specs=out_specs,
        out_shape=out_shape,
    )(template, local_feat, global_feat[:, None, :],
      w1t, w1l, w1g, row2(b1),
      w2, row2(b2),
      wst, wsl, wsg, row2(bskip),
      wot, row2(bout),
      wm1t, wm1l, wm1g, row2(bm1),
      wm2, row2(bm2), wm3, row2(bm3))
    return disp, mat[..., 0]


def kernel(template, surf_xyz, global_feat, point_feat, W1, b1, W2, b2,
           Wskip, bskip, Wout, bout, Wm1, bm1, Wm2, bm2, Wm3, bm3):
    B, T, _ = template.shape
    S = surf_xyz.shape[1]
    LD = point_feat.shape[2]
    params = (W1, b1, W2, b2, Wskip, bskip, Wout, bout,
              Wm1, bm1, Wm2, bm2, Wm3, bm3)

    surf_t = surf_xyz.transpose(0, 2, 1)

    tops = [_topk(template[b:b + 1], surf_t[b:b + 1], RT=256)
            for b in range(B)]
    locals_ = [_sc_gather(point_feat[b], idx.reshape(-1), w.reshape(-1),
                          T, LD)[None]
               for b, (idx, w) in enumerate(tops)]
    outs = [_mlp(template[b:b + 1], locals_[b], global_feat[b:b + 1],
                 params, RT=512)
            for b in range(B)]
    disp = jnp.concatenate([o[0] for o in outs], axis=0)
    mat = jnp.concatenate([o[1] for o in outs], axis=0)
    return disp, mat
